# Initial kernel scaffold; baseline (speedup 1.0000x reference)
import jax
import jax.numpy as jnp
from jax.experimental import pallas as pl


def kernel(in_feat, edge_index, W1, attn_l1, attn_r1, b1, W2, attn_l2, attn_r2, b2):
    raise NotImplementedError("write your pallas kernel here")


# speedup vs baseline: 11.0787x; 11.0787x over previous
import jax
import jax.numpy as jnp
from jax import lax
from jax.experimental import pallas as pl
from jax.experimental.pallas import tpu as pltpu
from jax.experimental.pallas import tpu_sc as plsc

N = 10000
E = 160000
HEADS = 8
FPH = 64
NLANE = 16
CF = 128
KB = 64
NBATCH = E // KB
SROWS = 624
ZROWS = 640

_MESH = plsc.VectorSubcoreMesh(core_axis_name="c", subcore_axis_name="s")


def _f32(shape):
    return jax.ShapeDtypeStruct(shape, jnp.float32)


def _tc_proj(x, W, ALl, ALr, bn=2000):
    n, k = x.shape

    def body(x_ref, w_ref, al_ref, ar_ref, fa, fb, elp, erp):
        f = jnp.dot(x_ref[...], w_ref[...], preferred_element_type=jnp.float32)
        fa[0] = f[:, 0:128]
        fa[1] = f[:, 256:384]
        fb[0] = f[:, 128:256]
        fb[1] = f[:, 384:512]
        elp[...] = jnp.dot(f, al_ref[...], preferred_element_type=jnp.float32)
        erp[...] = jnp.dot(f, ar_ref[...], preferred_element_type=jnp.float32)

    return pl.pallas_call(
        body,
        grid=(n // bn,),
        in_specs=[
            pl.BlockSpec((bn, k), lambda i: (i, 0)),
            pl.BlockSpec(W.shape, lambda i: (0, 0)),
            pl.BlockSpec(ALl.shape, lambda i: (0, 0)),
            pl.BlockSpec(ALr.shape, lambda i: (0, 0)),
        ],
        out_specs=[pl.BlockSpec((2, bn, 128), lambda i: (0, i, 0))] * 2
        + [pl.BlockSpec((bn, 128), lambda i: (i, 0))] * 2,
        out_shape=[_f32((2, n, 128))] * 2 + [_f32((n, 128))] * 2,
    )(x, W, ALl, ALr)


def _tc_elu_proj(oa, ob, b, W, ALl, ALr, bn=2000):
    n = oa.shape[1]

    def body(oa_ref, ob_ref, b_ref, w_ref, al_ref, ar_ref, fa, fb, elp, erp):
        parts = []
        for k, blk in enumerate((oa_ref[0], ob_ref[0], oa_ref[1], ob_ref[1])):
            v = blk + b_ref[:, 128 * k:128 * (k + 1)]
            parts.append(jnp.where(v > 0, v, jnp.exp(v) - 1.0))
        h = jnp.concatenate(parts, axis=1)
        f = jnp.dot(h, w_ref[...], preferred_element_type=jnp.float32)
        fa[0] = f[:, 0:128]
        fa[1] = f[:, 256:384]
        fb[0] = f[:, 128:256]
        fb[1] = f[:, 384:512]
        elp[...] = jnp.dot(f, al_ref[...], preferred_element_type=jnp.float32)
        erp[...] = jnp.dot(f, ar_ref[...], preferred_element_type=jnp.float32)

    return pl.pallas_call(
        body,
        grid=(n // bn,),
        in_specs=[pl.BlockSpec((2, bn, 128), lambda i: (0, i, 0))] * 2
        + [
            pl.BlockSpec((1, 512), lambda i: (0, 0)),
            pl.BlockSpec(W.shape, lambda i: (0, 0)),
            pl.BlockSpec(ALl.shape, lambda i: (0, 0)),
            pl.BlockSpec(ALr.shape, lambda i: (0, 0)),
        ],
        out_specs=[pl.BlockSpec((2, bn, 128), lambda i: (0, i, 0))] * 2
        + [pl.BlockSpec((bn, 128), lambda i: (i, 0))] * 2,
        out_shape=[_f32((2, n, 128))] * 2 + [_f32((n, 128))] * 2,
    )(oa, ob, b, W, ALl, ALr)


def _tc_elu_out(oa, ob, b, bn=2000):
    n = oa.shape[1]

    def body(oa_ref, ob_ref, b_ref, out):
        parts = []
        for k, blk in enumerate((oa_ref[0], ob_ref[0], oa_ref[1], ob_ref[1])):
            v = blk + b_ref[:, 128 * k:128 * (k + 1)]
            parts.append(jnp.where(v > 0, v, jnp.exp(v) - 1.0))
        out[...] = jnp.concatenate(parts, axis=1)

    return pl.pallas_call(
        body,
        grid=(n // bn,),
        in_specs=[pl.BlockSpec((2, bn, 128), lambda i: (0, i, 0))] * 2
        + [pl.BlockSpec((1, 512), lambda i: (0, 0))],
        out_specs=pl.BlockSpec((bn, 512), lambda i: (i, 0)),
        out_shape=_f32((n, 512)),
    )(oa, ob, b)


def _tc_alpha_rep(alpha, R, be=8000):
    e = alpha.shape[0]

    def body(a_ref, r_ref, ala, alb):
        ar = jnp.dot(a_ref[...], r_ref[...], preferred_element_type=jnp.float32)
        ala[0] = ar[:, 0:32]
        ala[1] = ar[:, 64:96]
        alb[0] = ar[:, 32:64]
        alb[1] = ar[:, 96:128]

    return pl.pallas_call(
        body,
        grid=(e // be,),
        in_specs=[
            pl.BlockSpec((be, NLANE), lambda i: (i, 0)),
            pl.BlockSpec(R.shape, lambda i: (0, 0)),
        ],
        out_specs=[pl.BlockSpec((2, be, 32), lambda i: (0, i, 0))] * 2,
        out_shape=[_f32((2, e, 32))] * 2,
    )(alpha, R)


def _nb(idx, total, parts):
    return jnp.where(idx < total - parts * (total // parts),
                     total // parts + 1, total // parts)


def _sc_softmax_num(elp, erp, src, dst, zeros):
    def body(elp_h, erp_h, src_h, dst_h, z_h, ex_h, dp0_h, dp1_h,
             acc, sidx, didx, g1, g2, exb, exb16, sem):
        cid = lax.axis_index("c")
        sid = lax.axis_index("s")
        wid = cid * 16 + sid
        row0 = sid * SROWS

        pltpu.sync_copy(z_h.at[pl.ds(0, SROWS)], acc.at[pl.ds(row0, SROWS)])

        @pl.when(sid == 15)
        def _():
            pltpu.sync_copy(z_h.at[pl.ds(0, 16)], acc.at[pl.ds(16 * SROWS, 16)])

        pltpu.sync_copy(z_h.at[pl.ds(0, KB)], exb)
        plsc.subcore_barrier()

        def batch(t, _):
            start = (wid + 32 * t) * KB
            pltpu.sync_copy(src_h.at[pl.ds(start, KB)], sidx)
            pltpu.sync_copy(dst_h.at[pl.ds(start, KB)], didx)
            c1 = pltpu.async_copy(elp_h.at[sidx], g1, sem)
            c2 = pltpu.async_copy(erp_h.at[didx], g2, sem)
            c1.wait()
            c2.wait()

            @plsc.parallel_loop(0, KB, unroll=4)
            def _(i):
                e = g1[i, pl.ds(0, NLANE)] + g2[i, pl.ds(0, NLANE)]
                e = jnp.maximum(e, 0.2 * e)
                ex = jnp.exp(e)
                exb[i, pl.ds(0, NLANE)] = ex
                exb16[i, :] = ex

            pltpu.sync_copy(exb16, ex_h.at[pl.ds(start, KB)])
            pltpu.sync_copy(exb, acc.at[didx], add=True)
            return 0

        lax.fori_loop(0, _nb(wid, NBATCH, 32), batch, 0)
        plsc.subcore_barrier()

        @pl.when(cid == 0)
        def _():
            pltpu.sync_copy(acc.at[pl.ds(row0, SROWS)], dp0_h.at[pl.ds(row0, SROWS)])

            @pl.when(sid == 15)
            def _():
                pltpu.sync_copy(acc.at[pl.ds(16 * SROWS, 16)],
                                dp0_h.at[pl.ds(16 * SROWS, 16)])

        @pl.when(cid == 1)
        def _():
            pltpu.sync_copy(acc.at[pl.ds(row0, SROWS)], dp1_h.at[pl.ds(row0, SROWS)])

            @pl.when(sid == 15)
            def _():
                pltpu.sync_copy(acc.at[pl.ds(16 * SROWS, 16)],
                                dp1_h.at[pl.ds(16 * SROWS, 16)])

    return pl.kernel(
        body,
        out_type=(_f32((E, NLANE)), _f32((N, 128)), _f32((N, 128))),
        mesh=_MESH,
        scratch_types=[
            pltpu.VMEM_SHARED((N, 128), jnp.float32),
            pltpu.VMEM((KB,), jnp.int32),
            pltpu.VMEM((KB,), jnp.int32),
            pltpu.VMEM((KB, 128), jnp.float32),
            pltpu.VMEM((KB, 128), jnp.float32),
            pltpu.VMEM((KB, 128), jnp.float32),
            pltpu.VMEM((KB, NLANE), jnp.float32),
            pltpu.SemaphoreType.DMA,
        ],
    )(elp, erp, src, dst, zeros)


def _sc_alpha(ex, dp0, dp1, dst):
    def body(ex_h, dp0_h, dp1_h, dst_h, al_h, didx, exb, d0b, d1b, alb, sem):
        cid = lax.axis_index("c")
        sid = lax.axis_index("s")
        wid = cid * 16 + sid

        def batch(t, _):
            start = (wid + 32 * t) * KB
            pltpu.sync_copy(dst_h.at[pl.ds(start, KB)], didx)
            pltpu.sync_copy(ex_h.at[pl.ds(start, KB)], exb)
            c1 = pltpu.async_copy(dp0_h.at[didx], d0b, sem)
            c2 = pltpu.async_copy(dp1_h.at[didx], d1b, sem)
            c1.wait()
            c2.wait()

            @plsc.parallel_loop(0, KB, unroll=4)
            def _(i):
                den = d0b[i, pl.ds(0, NLANE)] + d1b[i, pl.ds(0, NLANE)] + 1e-9
                alb[i, :] = exb[i, :] / den

            pltpu.sync_copy(alb, al_h.at[pl.ds(start, KB)])
            return 0

        lax.fori_loop(0, _nb(wid, NBATCH, 32), batch, 0)

    return pl.kernel(
        body,
        out_type=_f32((E, NLANE)),
        mesh=_MESH,
        scratch_types=[
            pltpu.VMEM((KB,), jnp.int32),
            pltpu.VMEM((KB, NLANE), jnp.float32),
            pltpu.VMEM((KB, 128), jnp.float32),
            pltpu.VMEM((KB, 128), jnp.float32),
            pltpu.VMEM((KB, NLANE), jnp.float32),
            pltpu.SemaphoreType.DMA,
        ],
    )(ex, dp0, dp1, dst)


def _sc_aggregate(fa, fb, ala, alb_, src, dst, zeros):
    def body(fa_h, fb_h, ala_h, alb_h, src_h, dst_h, z_h, oa_h, ob_h,
             acc, sidx, didx, gb, alb, sb, sem):
        cid = lax.axis_index("c")
        sid = lax.axis_index("s")
        row0 = sid * SROWS
        rowoff = cid * N
        aloff = cid * E
        nb = _nb(sid, NBATCH, 16)

        for j, (f_h, al_h, o_h) in enumerate(
                ((fa_h, ala_h, oa_h), (fb_h, alb_h, ob_h))):
            pltpu.sync_copy(z_h.at[pl.ds(0, SROWS)], acc.at[pl.ds(row0, SROWS)])

            @pl.when(sid == 15)
            def _():
                pltpu.sync_copy(z_h.at[pl.ds(0, 16)], acc.at[pl.ds(16 * SROWS, 16)])

            plsc.subcore_barrier()

            def batch(t, _):
                start = (sid + 16 * t) * KB
                pltpu.sync_copy(src_h.at[pl.ds(start, KB)], sidx)
                pltpu.sync_copy(dst_h.at[pl.ds(start, KB)], didx)
                pltpu.sync_copy(al_h.at[pl.ds(aloff + start, KB)], alb)
                roff = jnp.full((NLANE,), rowoff, jnp.int32)
                for q in range(KB // NLANE):
                    sidx[pl.ds(NLANE * q, NLANE)] = (
                        sidx[pl.ds(NLANE * q, NLANE)] + roff)
                pltpu.async_copy(f_h.at[sidx], gb, sem).wait()

                @plsc.parallel_loop(0, KB, unroll=2)
                def _(i):
                    a0 = alb[i, pl.ds(0, NLANE)]
                    a1 = alb[i, pl.ds(NLANE, NLANE)]
                    for k in range(8):
                        a = a0 if k < 4 else a1
                        sb[i, pl.ds(16 * k, 16)] = gb[i, pl.ds(16 * k, 16)] * a

                pltpu.sync_copy(sb, acc.at[didx], add=True)
                return 0

            lax.fori_loop(0, nb, batch, 0)
            plsc.subcore_barrier()
            pltpu.sync_copy(acc.at[pl.ds(row0, SROWS)],
                            o_h.at[pl.ds(rowoff + row0, SROWS)])

            @pl.when(sid == 15)
            def _():
                pltpu.sync_copy(acc.at[pl.ds(16 * SROWS, 16)],
                                o_h.at[pl.ds(rowoff + 16 * SROWS, 16)])

            plsc.subcore_barrier()

    return pl.kernel(
        body,
        out_type=(_f32((2 * N, CF)), _f32((2 * N, CF))),
        mesh=_MESH,
        scratch_types=[
            pltpu.VMEM_SHARED((N, CF), jnp.float32),
            pltpu.VMEM((KB,), jnp.int32),
            pltpu.VMEM((KB,), jnp.int32),
            pltpu.VMEM((KB, CF), jnp.float32),
            pltpu.VMEM((KB, 32), jnp.float32),
            pltpu.VMEM((KB, CF), jnp.float32),
            pltpu.SemaphoreType.DMA,
        ],
    )(fa, fb, ala, alb_, src, dst, zeros)


def _attn_mats(attn_l, attn_r):
    eye = jnp.eye(HEADS, dtype=jnp.float32)
    Ml = (attn_l[:, :, None] * eye[:, None, :]).reshape(HEADS * FPH, HEADS)
    Mr = (attn_r[:, :, None] * eye[:, None, :]).reshape(HEADS * FPH, HEADS)
    z = jnp.zeros((HEADS * FPH, 128 - HEADS), jnp.float32)
    return jnp.concatenate([Ml, z], 1), jnp.concatenate([Mr, z], 1)


def _rep_mat():
    j = jnp.arange(128)
    head = (j // 32) * 2 + (j % 32) // 16
    return (jnp.arange(NLANE)[:, None] == head[None, :]).astype(jnp.float32)


def _gat_sc_layer(fa, fb, elp, erp, src, dst, zeros, R):
    ex, dp0, dp1 = _sc_softmax_num(elp, erp, src, dst, zeros)
    alpha = _sc_alpha(ex, dp0, dp1, dst)
    ala, alb = _tc_alpha_rep(alpha, R)
    oa, ob = _sc_aggregate(fa.reshape(2 * N, CF), fb.reshape(2 * N, CF),
                           ala.reshape(2 * E, 32), alb.reshape(2 * E, 32),
                           src, dst, zeros)
    return oa.reshape(2, N, CF), ob.reshape(2, N, CF)


@jax.jit
def kernel(in_feat, edge_index, W1, attn_l1, attn_r1, b1, W2, attn_l2, attn_r2, b2):
    src = edge_index[0]
    dst = edge_index[1]
    ALl1, ALr1 = _attn_mats(attn_l1, attn_r1)
    ALl2, ALr2 = _attn_mats(attn_l2, attn_r2)
    b1r = b1.reshape(1, -1)
    b2r = b2.reshape(1, -1)
    zeros = jnp.zeros((ZROWS, 128), jnp.float32)
    R = _rep_mat()

    fa, fb, elp, erp = _tc_proj(in_feat, W1, ALl1, ALr1)
    oa, ob = _gat_sc_layer(fa, fb, elp, erp, src, dst, zeros, R)

    fa, fb, elp, erp = _tc_elu_proj(oa, ob, b1r, W2, ALl2, ALr2)
    oa, ob = _gat_sc_layer(fa, fb, elp, erp, src, dst, zeros, R)

    return _tc_elu_out(oa, ob, b2r)


# speedup vs baseline: 12.6472x; 1.1416x over previous
import jax
import jax.numpy as jnp
from jax import lax
from jax.experimental import pallas as pl
from jax.experimental.pallas import tpu as pltpu
from jax.experimental.pallas import tpu_sc as plsc

N = 10000
E = 160000
HEADS = 8
FPH = 64
NLANE = 16
CF = 128
KB = 64
NBATCH = E // KB
SROWS = 624
ZROWS = 640

_MESH = plsc.VectorSubcoreMesh(core_axis_name="c", subcore_axis_name="s")


def _f32(shape):
    return jax.ShapeDtypeStruct(shape, jnp.float32)


def _tc_proj(x, W, ALl, ALr, bn=2000):
    n, k = x.shape

    def body(x_ref, w_ref, al_ref, ar_ref, fa, fb, elp, erp):
        f = jnp.dot(x_ref[...], w_ref[...], preferred_element_type=jnp.float32)
        fa[0] = f[:, 0:128]
        fa[1] = f[:, 256:384]
        fb[0] = f[:, 128:256]
        fb[1] = f[:, 384:512]
        elp[...] = jnp.dot(f, al_ref[...], preferred_element_type=jnp.float32)
        erp[...] = jnp.dot(f, ar_ref[...], preferred_element_type=jnp.float32)

    return pl.pallas_call(
        body,
        grid=(n // bn,),
        in_specs=[
            pl.BlockSpec((bn, k), lambda i: (i, 0)),
            pl.BlockSpec(W.shape, lambda i: (0, 0)),
            pl.BlockSpec(ALl.shape, lambda i: (0, 0)),
            pl.BlockSpec(ALr.shape, lambda i: (0, 0)),
        ],
        out_specs=[pl.BlockSpec((2, bn, 128), lambda i: (0, i, 0))] * 2
        + [pl.BlockSpec((bn, 128), lambda i: (i, 0))] * 2,
        out_shape=[_f32((2, n, 128))] * 2 + [_f32((n, 128))] * 2,
    )(x, W, ALl, ALr)


def _tc_elu_proj(oa, ob, b, W, ALl, ALr, bn=2000):
    n = oa.shape[1]

    def body(oa_ref, ob_ref, b_ref, w_ref, al_ref, ar_ref, fa, fb, elp, erp):
        parts = []
        for k, blk in enumerate((oa_ref[0], ob_ref[0], oa_ref[1], ob_ref[1])):
            v = blk + b_ref[:, 128 * k:128 * (k + 1)]
            parts.append(jnp.where(v > 0, v, jnp.exp(v) - 1.0))
        h = jnp.concatenate(parts, axis=1)
        f = jnp.dot(h, w_ref[...], preferred_element_type=jnp.float32)
        fa[0] = f[:, 0:128]
        fa[1] = f[:, 256:384]
        fb[0] = f[:, 128:256]
        fb[1] = f[:, 384:512]
        elp[...] = jnp.dot(f, al_ref[...], preferred_element_type=jnp.float32)
        erp[...] = jnp.dot(f, ar_ref[...], preferred_element_type=jnp.float32)

    return pl.pallas_call(
        body,
        grid=(n // bn,),
        in_specs=[pl.BlockSpec((2, bn, 128), lambda i: (0, i, 0))] * 2
        + [
            pl.BlockSpec((1, 512), lambda i: (0, 0)),
            pl.BlockSpec(W.shape, lambda i: (0, 0)),
            pl.BlockSpec(ALl.shape, lambda i: (0, 0)),
            pl.BlockSpec(ALr.shape, lambda i: (0, 0)),
        ],
        out_specs=[pl.BlockSpec((2, bn, 128), lambda i: (0, i, 0))] * 2
        + [pl.BlockSpec((bn, 128), lambda i: (i, 0))] * 2,
        out_shape=[_f32((2, n, 128))] * 2 + [_f32((n, 128))] * 2,
    )(oa, ob, b, W, ALl, ALr)


def _tc_elu_out(oa, ob, b, bn=2000):
    n = oa.shape[1]

    def body(oa_ref, ob_ref, b_ref, out):
        parts = []
        for k, blk in enumerate((oa_ref[0], ob_ref[0], oa_ref[1], ob_ref[1])):
            v = blk + b_ref[:, 128 * k:128 * (k + 1)]
            parts.append(jnp.where(v > 0, v, jnp.exp(v) - 1.0))
        out[...] = jnp.concatenate(parts, axis=1)

    return pl.pallas_call(
        body,
        grid=(n // bn,),
        in_specs=[pl.BlockSpec((2, bn, 128), lambda i: (0, i, 0))] * 2
        + [pl.BlockSpec((1, 512), lambda i: (0, 0))],
        out_specs=pl.BlockSpec((bn, 512), lambda i: (i, 0)),
        out_shape=_f32((n, 512)),
    )(oa, ob, b)


def _tc_alpha_rep(alpha, R, be=8000):
    e = alpha.shape[0]

    def body(a_ref, r_ref, ala, alb):
        ar = jnp.dot(a_ref[...], r_ref[...], preferred_element_type=jnp.float32)
        ala[0] = ar[:, 0:32]
        ala[1] = ar[:, 64:96]
        alb[0] = ar[:, 32:64]
        alb[1] = ar[:, 96:128]

    return pl.pallas_call(
        body,
        grid=(e // be,),
        in_specs=[
            pl.BlockSpec((be, NLANE), lambda i: (i, 0)),
            pl.BlockSpec(R.shape, lambda i: (0, 0)),
        ],
        out_specs=[pl.BlockSpec((2, be, 32), lambda i: (0, i, 0))] * 2,
        out_shape=[_f32((2, e, 32))] * 2,
    )(alpha, R)


def _nb(idx, total, parts):
    return jnp.where(idx < total - parts * (total // parts),
                     total // parts + 1, total // parts)


def _sc_softmax_num(elp, erp, src, dst, zeros):
    def body(elp_h, erp_h, src_h, dst_h, z_h, ex_h, dp0_h, dp1_h,
             acc, sidx, didx, g1, g2, exb, exb16, sem):
        cid = lax.axis_index("c")
        sid = lax.axis_index("s")
        wid = cid * 16 + sid
        row0 = sid * SROWS

        pltpu.sync_copy(z_h.at[pl.ds(0, SROWS)], acc.at[pl.ds(row0, SROWS)])

        @pl.when(sid == 15)
        def _():
            pltpu.sync_copy(z_h.at[pl.ds(0, 16)], acc.at[pl.ds(16 * SROWS, 16)])

        pltpu.sync_copy(z_h.at[pl.ds(0, KB)], exb)
        plsc.subcore_barrier()

        def batch(t, _):
            start = (wid + 32 * t) * KB
            pltpu.sync_copy(src_h.at[pl.ds(start, KB)], sidx)
            pltpu.sync_copy(dst_h.at[pl.ds(start, KB)], didx)
            c1 = pltpu.async_copy(elp_h.at[sidx], g1, sem)
            c2 = pltpu.async_copy(erp_h.at[didx], g2, sem)
            c1.wait()
            c2.wait()

            @plsc.parallel_loop(0, KB, unroll=4)
            def _(i):
                e = g1[i, pl.ds(0, NLANE)] + g2[i, pl.ds(0, NLANE)]
                e = jnp.maximum(e, 0.2 * e)
                ex = jnp.exp(e)
                exb[i, pl.ds(0, NLANE)] = ex
                exb16[i, :] = ex

            pltpu.sync_copy(exb16, ex_h.at[pl.ds(start, KB)])
            pltpu.sync_copy(exb, acc.at[didx], add=True)
            return 0

        lax.fori_loop(0, _nb(wid, NBATCH, 32), batch, 0)
        plsc.subcore_barrier()

        @pl.when(cid == 0)
        def _():
            pltpu.sync_copy(acc.at[pl.ds(row0, SROWS)], dp0_h.at[pl.ds(row0, SROWS)])

            @pl.when(sid == 15)
            def _():
                pltpu.sync_copy(acc.at[pl.ds(16 * SROWS, 16)],
                                dp0_h.at[pl.ds(16 * SROWS, 16)])

        @pl.when(cid == 1)
        def _():
            pltpu.sync_copy(acc.at[pl.ds(row0, SROWS)], dp1_h.at[pl.ds(row0, SROWS)])

            @pl.when(sid == 15)
            def _():
                pltpu.sync_copy(acc.at[pl.ds(16 * SROWS, 16)],
                                dp1_h.at[pl.ds(16 * SROWS, 16)])

    return pl.kernel(
        body,
        out_type=(_f32((E, NLANE)), _f32((N, 128)), _f32((N, 128))),
        mesh=_MESH,
        scratch_types=[
            pltpu.VMEM_SHARED((N, 128), jnp.float32),
            pltpu.VMEM((KB,), jnp.int32),
            pltpu.VMEM((KB,), jnp.int32),
            pltpu.VMEM((KB, 128), jnp.float32),
            pltpu.VMEM((KB, 128), jnp.float32),
            pltpu.VMEM((KB, 128), jnp.float32),
            pltpu.VMEM((KB, NLANE), jnp.float32),
            pltpu.SemaphoreType.DMA,
        ],
    )(elp, erp, src, dst, zeros)


def _sc_alpha(ex, dp0, dp1, dst):
    def body(ex_h, dp0_h, dp1_h, dst_h, al_h, didx, exb, d0b, d1b, alb, sem):
        cid = lax.axis_index("c")
        sid = lax.axis_index("s")
        wid = cid * 16 + sid

        def batch(t, _):
            start = (wid + 32 * t) * KB
            pltpu.sync_copy(dst_h.at[pl.ds(start, KB)], didx)
            pltpu.sync_copy(ex_h.at[pl.ds(start, KB)], exb)
            c1 = pltpu.async_copy(dp0_h.at[didx], d0b, sem)
            c2 = pltpu.async_copy(dp1_h.at[didx], d1b, sem)
            c1.wait()
            c2.wait()

            @plsc.parallel_loop(0, KB, unroll=4)
            def _(i):
                den = d0b[i, pl.ds(0, NLANE)] + d1b[i, pl.ds(0, NLANE)] + 1e-9
                alb[i, :] = exb[i, :] / den

            pltpu.sync_copy(alb, al_h.at[pl.ds(start, KB)])
            return 0

        lax.fori_loop(0, _nb(wid, NBATCH, 32), batch, 0)

    return pl.kernel(
        body,
        out_type=_f32((E, NLANE)),
        mesh=_MESH,
        scratch_types=[
            pltpu.VMEM((KB,), jnp.int32),
            pltpu.VMEM((KB, NLANE), jnp.float32),
            pltpu.VMEM((KB, 128), jnp.float32),
            pltpu.VMEM((KB, 128), jnp.float32),
            pltpu.VMEM((KB, NLANE), jnp.float32),
            pltpu.SemaphoreType.DMA,
        ],
    )(ex, dp0, dp1, dst)


def _sc_aggregate(fa, fb, ala, alb_, src2, dst, zeros):
    def body(fa_h, fb_h, ala_h, alb_h, src2_h, dst_h, z_h, oa_h, ob_h,
             acc, sidx0, sidx1, didx0, didx1, gb0, gb1, ab0, ab1, sb0, sb1,
             gsem0, gsem1, ssem0, ssem1):
        cid = lax.axis_index("c")
        sid = lax.axis_index("s")
        row0 = sid * SROWS
        rowoff = cid * N
        eoff = cid * E
        nb = _nb(sid, NBATCH, 16)
        sets = ((sidx0, didx0, gb0, ab0, sb0, gsem0, ssem0),
                (sidx1, didx1, gb1, ab1, sb1, gsem1, ssem1))

        for j, (f_h, al_h, o_h) in enumerate(
                ((fa_h, ala_h, oa_h), (fb_h, alb_h, ob_h))):
            pltpu.sync_copy(z_h.at[pl.ds(0, SROWS)], acc.at[pl.ds(row0, SROWS)])

            @pl.when(sid == 15)
            def _():
                pltpu.sync_copy(z_h.at[pl.ds(0, 16)], acc.at[pl.ds(16 * SROWS, 16)])

            plsc.subcore_barrier()

            def load_idx(t, S):
                start = (sid + 16 * t) * KB
                pltpu.sync_copy(src2_h.at[pl.ds(eoff + start, KB)], S[0])
                pltpu.sync_copy(dst_h.at[pl.ds(start, KB)], S[1])
                pltpu.sync_copy(al_h.at[pl.ds(eoff + start, KB)], S[3])

            load_idx(0, sets[0])
            pltpu.async_copy(f_h.at[sidx0], gb0, gsem0)

            def pair(u, _):
                for p in (0, 1):
                    t = 2 * u + p
                    S = sets[p]
                    S2 = sets[1 - p]

                    @pl.when(t < nb)
                    def _():
                        pltpu.make_async_copy(f_h.at[S[0]], S[2], S[5]).wait()

                        @pl.when(t >= 1)
                        def _():
                            pltpu.make_async_copy(S2[4], acc.at[S2[1]], S2[6]).wait()

                        @pl.when(t + 1 < nb)
                        def _():
                            load_idx(t + 1, S2)
                            pltpu.async_copy(f_h.at[S2[0]], S2[2], S2[5])

                        gb, ab, sb = S[2], S[3], S[4]

                        @plsc.parallel_loop(0, KB, unroll=2)
                        def _(i):
                            a0 = ab[i, pl.ds(0, NLANE)]
                            a1 = ab[i, pl.ds(NLANE, NLANE)]
                            for k in range(8):
                                a = a0 if k < 4 else a1
                                sb[i, pl.ds(16 * k, 16)] = gb[i, pl.ds(16 * k, 16)] * a

                        pltpu.async_copy(S[4], acc.at[S[1]], S[6], add=True)

                return 0

            lax.fori_loop(0, (nb + 1) // 2, pair, 0)

            @pl.when(lax.rem(nb - 1, 2) == 0)
            def _():
                pltpu.make_async_copy(sb0, acc.at[didx0], ssem0).wait()

            @pl.when(lax.rem(nb - 1, 2) == 1)
            def _():
                pltpu.make_async_copy(sb1, acc.at[didx1], ssem1).wait()

            plsc.subcore_barrier()
            pltpu.sync_copy(acc.at[pl.ds(row0, SROWS)],
                            o_h.at[pl.ds(rowoff + row0, SROWS)])

            @pl.when(sid == 15)
            def _():
                pltpu.sync_copy(acc.at[pl.ds(16 * SROWS, 16)],
                                o_h.at[pl.ds(rowoff + 16 * SROWS, 16)])

            plsc.subcore_barrier()

    return pl.kernel(
        body,
        out_type=(_f32((2 * N, CF)), _f32((2 * N, CF))),
        mesh=_MESH,
        scratch_types=[
            pltpu.VMEM_SHARED((N, CF), jnp.float32),
            pltpu.VMEM((KB,), jnp.int32),
            pltpu.VMEM((KB,), jnp.int32),
            pltpu.VMEM((KB,), jnp.int32),
            pltpu.VMEM((KB,), jnp.int32),
            pltpu.VMEM((KB, CF), jnp.float32),
            pltpu.VMEM((KB, CF), jnp.float32),
            pltpu.VMEM((KB, 32), jnp.float32),
            pltpu.VMEM((KB, 32), jnp.float32),
            pltpu.VMEM((KB, CF), jnp.float32),
            pltpu.VMEM((KB, CF), jnp.float32),
            pltpu.SemaphoreType.DMA,
            pltpu.SemaphoreType.DMA,
            pltpu.SemaphoreType.DMA,
            pltpu.SemaphoreType.DMA,
        ],
    )(fa, fb, ala, alb_, src2, dst, zeros)


def _attn_mats(attn_l, attn_r):
    eye = jnp.eye(HEADS, dtype=jnp.float32)
    Ml = (attn_l[:, :, None] * eye[:, None, :]).reshape(HEADS * FPH, HEADS)
    Mr = (attn_r[:, :, None] * eye[:, None, :]).reshape(HEADS * FPH, HEADS)
    z = jnp.zeros((HEADS * FPH, 128 - HEADS), jnp.float32)
    return jnp.concatenate([Ml, z], 1), jnp.concatenate([Mr, z], 1)


def _rep_mat():
    j = jnp.arange(128)
    head = (j // 32) * 2 + (j % 32) // 16
    return (jnp.arange(NLANE)[:, None] == head[None, :]).astype(jnp.float32)


def _gat_sc_layer(fa, fb, elp, erp, src, src2, dst, zeros, R):
    ex, dp0, dp1 = _sc_softmax_num(elp, erp, src, dst, zeros)
    alpha = _sc_alpha(ex, dp0, dp1, dst)
    ala, alb = _tc_alpha_rep(alpha, R)
    oa, ob = _sc_aggregate(fa.reshape(2 * N, CF), fb.reshape(2 * N, CF),
                           ala.reshape(2 * E, 32), alb.reshape(2 * E, 32),
                           src2, dst, zeros)
    return oa.reshape(2, N, CF), ob.reshape(2, N, CF)


@jax.jit
def kernel(in_feat, edge_index, W1, attn_l1, attn_r1, b1, W2, attn_l2, attn_r2, b2):
    src = edge_index[0]
    dst = edge_index[1]
    src2 = jnp.concatenate([src, src + N])
    ALl1, ALr1 = _attn_mats(attn_l1, attn_r1)
    ALl2, ALr2 = _attn_mats(attn_l2, attn_r2)
    b1r = b1.reshape(1, -1)
    b2r = b2.reshape(1, -1)
    zeros = jnp.zeros((ZROWS, 128), jnp.float32)
    R = _rep_mat()

    fa, fb, elp, erp = _tc_proj(in_feat, W1, ALl1, ALr1)
    oa, ob = _gat_sc_layer(fa, fb, elp, erp, src, src2, dst, zeros, R)

    fa, fb, elp, erp = _tc_elu_proj(oa, ob, b1r, W2, ALl2, ALr2)
    oa, ob = _gat_sc_layer(fa, fb, elp, erp, src, src2, dst, zeros, R)

    return _tc_elu_out(oa, ob, b2r)


# speedup vs baseline: 15.8538x; 1.2535x over previous
import jax
import jax.numpy as jnp
from jax import lax
from jax.experimental import pallas as pl
from jax.experimental.pallas import tpu as pltpu
from jax.experimental.pallas import tpu_sc as plsc

N = 10000
E = 160000
HEADS = 8
FPH = 64
NLANE = 16
CF = 128
KB = 64
NBATCH = E // KB
SROWS = 624
ZROWS = 640

_MESH = plsc.VectorSubcoreMesh(core_axis_name="c", subcore_axis_name="s")


def _f32(shape):
    return jax.ShapeDtypeStruct(shape, jnp.float32)


def _tc_proj(x, W, ALl, ALr, bn=2000):
    n, k = x.shape

    def body(x_ref, w_ref, al_ref, ar_ref, fa, fb, elp, erp):
        f = jnp.dot(x_ref[...], w_ref[...], preferred_element_type=jnp.float32)
        fa[0] = f[:, 0:128]
        fa[1] = f[:, 256:384]
        fb[0] = f[:, 128:256]
        fb[1] = f[:, 384:512]
        elp[...] = jnp.dot(f, al_ref[...], preferred_element_type=jnp.float32)
        erp[...] = jnp.dot(f, ar_ref[...], preferred_element_type=jnp.float32)

    return pl.pallas_call(
        body,
        grid=(n // bn,),
        in_specs=[
            pl.BlockSpec((bn, k), lambda i: (i, 0)),
            pl.BlockSpec(W.shape, lambda i: (0, 0)),
            pl.BlockSpec(ALl.shape, lambda i: (0, 0)),
            pl.BlockSpec(ALr.shape, lambda i: (0, 0)),
        ],
        out_specs=[pl.BlockSpec((2, bn, 128), lambda i: (0, i, 0))] * 2
        + [pl.BlockSpec((bn, 128), lambda i: (i, 0))] * 2,
        out_shape=[_f32((2, n, 128))] * 2 + [_f32((n, 128))] * 2,
    )(x, W, ALl, ALr)


def _tc_elu_proj(oa, ob, b, W, ALl, ALr, bn=2000):
    n = oa.shape[1]

    def body(oa_ref, ob_ref, b_ref, w_ref, al_ref, ar_ref, fa, fb, elp, erp):
        parts = []
        for k, blk in enumerate((oa_ref[0], ob_ref[0], oa_ref[1], ob_ref[1])):
            v = blk + b_ref[:, 128 * k:128 * (k + 1)]
            parts.append(jnp.where(v > 0, v, jnp.exp(v) - 1.0))
        h = jnp.concatenate(parts, axis=1)
        f = jnp.dot(h, w_ref[...], preferred_element_type=jnp.float32)
        fa[0] = f[:, 0:128]
        fa[1] = f[:, 256:384]
        fb[0] = f[:, 128:256]
        fb[1] = f[:, 384:512]
        elp[...] = jnp.dot(f, al_ref[...], preferred_element_type=jnp.float32)
        erp[...] = jnp.dot(f, ar_ref[...], preferred_element_type=jnp.float32)

    return pl.pallas_call(
        body,
        grid=(n // bn,),
        in_specs=[pl.BlockSpec((2, bn, 128), lambda i: (0, i, 0))] * 2
        + [
            pl.BlockSpec((1, 512), lambda i: (0, 0)),
            pl.BlockSpec(W.shape, lambda i: (0, 0)),
            pl.BlockSpec(ALl.shape, lambda i: (0, 0)),
            pl.BlockSpec(ALr.shape, lambda i: (0, 0)),
        ],
        out_specs=[pl.BlockSpec((2, bn, 128), lambda i: (0, i, 0))] * 2
        + [pl.BlockSpec((bn, 128), lambda i: (i, 0))] * 2,
        out_shape=[_f32((2, n, 128))] * 2 + [_f32((n, 128))] * 2,
    )(oa, ob, b, W, ALl, ALr)


def _tc_elu_out(oa, ob, b, bn=2000):
    n = oa.shape[1]

    def body(oa_ref, ob_ref, b_ref, out):
        parts = []
        for k, blk in enumerate((oa_ref[0], ob_ref[0], oa_ref[1], ob_ref[1])):
            v = blk + b_ref[:, 128 * k:128 * (k + 1)]
            parts.append(jnp.where(v > 0, v, jnp.exp(v) - 1.0))
        out[...] = jnp.concatenate(parts, axis=1)

    return pl.pallas_call(
        body,
        grid=(n // bn,),
        in_specs=[pl.BlockSpec((2, bn, 128), lambda i: (0, i, 0))] * 2
        + [pl.BlockSpec((1, 512), lambda i: (0, 0))],
        out_specs=pl.BlockSpec((bn, 512), lambda i: (i, 0)),
        out_shape=_f32((n, 512)),
    )(oa, ob, b)


def _tc_alpha_rep(alpha, R, be=8000):
    e = alpha.shape[0]

    def body(a_ref, r_ref, ala, alb):
        ar = jnp.dot(a_ref[...], r_ref[...], preferred_element_type=jnp.float32)
        ala[0] = ar[:, 0:32]
        ala[1] = ar[:, 64:96]
        alb[0] = ar[:, 32:64]
        alb[1] = ar[:, 96:128]

    return pl.pallas_call(
        body,
        grid=(e // be,),
        in_specs=[
            pl.BlockSpec((be, NLANE), lambda i: (i, 0)),
            pl.BlockSpec(R.shape, lambda i: (0, 0)),
        ],
        out_specs=[pl.BlockSpec((2, be, 32), lambda i: (0, i, 0))] * 2,
        out_shape=[_f32((2, e, 32))] * 2,
    )(alpha, R)


def _nb(idx, total, parts):
    return jnp.where(idx < total - parts * (total // parts),
                     total // parts + 1, total // parts)


def _sc_softmax_num(elp, erp, src, dst, zeros):
    def body(elp_h, erp_h, src_h, dst_h, z_h, ex_h, dp0_h, dp1_h,
             acc, sidx, didx, g1, g2, exb, exb16, sem):
        cid = lax.axis_index("c")
        sid = lax.axis_index("s")
        wid = cid * 16 + sid
        row0 = sid * SROWS

        pltpu.sync_copy(z_h.at[pl.ds(0, SROWS)], acc.at[pl.ds(row0, SROWS)])

        @pl.when(sid == 15)
        def _():
            pltpu.sync_copy(z_h.at[pl.ds(0, 16)], acc.at[pl.ds(16 * SROWS, 16)])

        pltpu.sync_copy(z_h.at[pl.ds(0, KB)], exb)
        plsc.subcore_barrier()

        def batch(t, _):
            start = (wid + 32 * t) * KB
            pltpu.sync_copy(src_h.at[pl.ds(start, KB)], sidx)
            pltpu.sync_copy(dst_h.at[pl.ds(start, KB)], didx)
            c1 = pltpu.async_copy(elp_h.at[sidx], g1, sem)
            c2 = pltpu.async_copy(erp_h.at[didx], g2, sem)
            c1.wait()
            c2.wait()

            @plsc.parallel_loop(0, KB, unroll=4)
            def _(i):
                e = g1[i, pl.ds(0, NLANE)] + g2[i, pl.ds(0, NLANE)]
                e = jnp.maximum(e, 0.2 * e)
                ex = jnp.exp(e)
                exb[i, pl.ds(0, NLANE)] = ex
                exb16[i, :] = ex

            pltpu.sync_copy(exb16, ex_h.at[pl.ds(start, KB)])
            pltpu.sync_copy(exb, acc.at[didx], add=True)
            return 0

        lax.fori_loop(0, _nb(wid, NBATCH, 32), batch, 0)
        plsc.subcore_barrier()

        @pl.when(cid == 0)
        def _():
            pltpu.sync_copy(acc.at[pl.ds(row0, SROWS)], dp0_h.at[pl.ds(row0, SROWS)])

            @pl.when(sid == 15)
            def _():
                pltpu.sync_copy(acc.at[pl.ds(16 * SROWS, 16)],
                                dp0_h.at[pl.ds(16 * SROWS, 16)])

        @pl.when(cid == 1)
        def _():
            pltpu.sync_copy(acc.at[pl.ds(row0, SROWS)], dp1_h.at[pl.ds(row0, SROWS)])

            @pl.when(sid == 15)
            def _():
                pltpu.sync_copy(acc.at[pl.ds(16 * SROWS, 16)],
                                dp1_h.at[pl.ds(16 * SROWS, 16)])

    return pl.kernel(
        body,
        out_type=(_f32((E, NLANE)), _f32((N, 128)), _f32((N, 128))),
        mesh=_MESH,
        scratch_types=[
            pltpu.VMEM_SHARED((N, 128), jnp.float32),
            pltpu.VMEM((KB,), jnp.int32),
            pltpu.VMEM((KB,), jnp.int32),
            pltpu.VMEM((KB, 128), jnp.float32),
            pltpu.VMEM((KB, 128), jnp.float32),
            pltpu.VMEM((KB, 128), jnp.float32),
            pltpu.VMEM((KB, NLANE), jnp.float32),
            pltpu.SemaphoreType.DMA,
        ],
    )(elp, erp, src, dst, zeros)


def _sc_alpha(ex, dp0, dp1, dst):
    KB2 = 128
    NB2 = E // KB2

    def body(ex_h, dp0_h, dp1_h, dst_h, al_h, didx, exb, d0b, d1b, alb, sem):
        cid = lax.axis_index("c")
        sid = lax.axis_index("s")
        wid = cid * 16 + sid

        def batch(t, _):
            start = (wid + 32 * t) * KB2
            pltpu.sync_copy(dst_h.at[pl.ds(start, KB2)], didx)
            pltpu.sync_copy(ex_h.at[pl.ds(start, KB2)], exb)
            c1 = pltpu.async_copy(dp0_h.at[didx], d0b, sem)
            c2 = pltpu.async_copy(dp1_h.at[didx], d1b, sem)
            c1.wait()
            c2.wait()

            @plsc.parallel_loop(0, KB2, unroll=4)
            def _(i):
                den = d0b[i, pl.ds(0, NLANE)] + d1b[i, pl.ds(0, NLANE)] + 1e-9
                alb[i, :] = exb[i, :] / den

            pltpu.sync_copy(alb, al_h.at[pl.ds(start, KB2)])
            return 0

        lax.fori_loop(0, _nb(wid, NB2, 32), batch, 0)

    return pl.kernel(
        body,
        out_type=_f32((E, NLANE)),
        mesh=_MESH,
        scratch_types=[
            pltpu.VMEM((128,), jnp.int32),
            pltpu.VMEM((128, NLANE), jnp.float32),
            pltpu.VMEM((128, 128), jnp.float32),
            pltpu.VMEM((128, 128), jnp.float32),
            pltpu.VMEM((128, NLANE), jnp.float32),
            pltpu.SemaphoreType.DMA,
        ],
    )(ex, dp0, dp1, dst)


def _sc_aggregate(fa, fb, ala, alb_, src2, dst, zeros):
    def body(fa_h, fb_h, ala_h, alb_h, src2_h, dst_h, z_h, oa_h, ob_h,
             acc, sidx0, sidx1, didx0, didx1, gb0, gb1, ab0, ab1, sb0, sb1,
             gsem0, gsem1, ssem0, ssem1, isem):
        cid = lax.axis_index("c")
        sid = lax.axis_index("s")
        row0 = sid * SROWS
        rowoff = cid * N
        eoff = cid * E
        nb = _nb(sid, NBATCH, 16)
        sets = ((sidx0, didx0, gb0, ab0, sb0, gsem0, ssem0),
                (sidx1, didx1, gb1, ab1, sb1, gsem1, ssem1))

        for j, (f_h, al_h, o_h) in enumerate(
                ((fa_h, ala_h, oa_h), (fb_h, alb_h, ob_h))):
            pltpu.sync_copy(z_h.at[pl.ds(0, SROWS)], acc.at[pl.ds(row0, SROWS)])

            @pl.when(sid == 15)
            def _():
                pltpu.sync_copy(z_h.at[pl.ds(0, 16)], acc.at[pl.ds(16 * SROWS, 16)])

            plsc.subcore_barrier()

            def load_idx(t, S):
                start = (sid + 16 * t) * KB
                pltpu.async_copy(src2_h.at[pl.ds(eoff + start, KB)], S[0], isem)
                pltpu.async_copy(dst_h.at[pl.ds(start, KB)], S[1], isem)
                pltpu.async_copy(al_h.at[pl.ds(eoff + start, KB)], S[3], isem)

            def drain_idx(t, S):
                start = (sid + 16 * t) * KB
                pltpu.make_async_copy(src2_h.at[pl.ds(eoff + start, KB)], S[0], isem).wait()
                pltpu.make_async_copy(dst_h.at[pl.ds(start, KB)], S[1], isem).wait()
                pltpu.make_async_copy(al_h.at[pl.ds(eoff + start, KB)], S[3], isem).wait()

            load_idx(0, sets[0])
            drain_idx(0, sets[0])
            pltpu.async_copy(f_h.at[sidx0], gb0, gsem0)

            def pair(u, _):
                for p in (0, 1):
                    t = 2 * u + p
                    S = sets[p]
                    S2 = sets[1 - p]

                    @pl.when(t < nb)
                    def _():
                        pltpu.make_async_copy(f_h.at[S[0]], S[2], S[5]).wait()

                        @pl.when(t >= 1)
                        def _():
                            pltpu.make_async_copy(S2[4], acc.at[S2[1]], S2[6]).wait()

                        @pl.when(t + 1 < nb)
                        def _():
                            load_idx(t + 1, S2)
                            drain_idx(t + 1, S2)
                            pltpu.async_copy(f_h.at[S2[0]], S2[2], S2[5])

                        gb, ab, sb = S[2], S[3], S[4]

                        @plsc.parallel_loop(0, KB, unroll=2)
                        def _(i):
                            a0 = ab[i, pl.ds(0, NLANE)]
                            a1 = ab[i, pl.ds(NLANE, NLANE)]
                            for k in range(8):
                                a = a0 if k < 4 else a1
                                sb[i, pl.ds(16 * k, 16)] = gb[i, pl.ds(16 * k, 16)] * a

                        pltpu.async_copy(S[4], acc.at[S[1]], S[6], add=True)

                return 0

            lax.fori_loop(0, (nb + 1) // 2, pair, 0)

            @pl.when(lax.rem(nb - 1, 2) == 0)
            def _():
                pltpu.make_async_copy(sb0, acc.at[didx0], ssem0).wait()

            @pl.when(lax.rem(nb - 1, 2) == 1)
            def _():
                pltpu.make_async_copy(sb1, acc.at[didx1], ssem1).wait()

            plsc.subcore_barrier()
            pltpu.sync_copy(acc.at[pl.ds(row0, SROWS)],
                            o_h.at[pl.ds(rowoff + row0, SROWS)])

            @pl.when(sid == 15)
            def _():
                pltpu.sync_copy(acc.at[pl.ds(16 * SROWS, 16)],
                                o_h.at[pl.ds(rowoff + 16 * SROWS, 16)])

            plsc.subcore_barrier()

    return pl.kernel(
        body,
        out_type=(_f32((2 * N, CF)), _f32((2 * N, CF))),
        mesh=_MESH,
        scratch_types=[
            pltpu.VMEM_SHARED((N, CF), jnp.float32),
            pltpu.VMEM((KB,), jnp.int32),
            pltpu.VMEM((KB,), jnp.int32),
            pltpu.VMEM((KB,), jnp.int32),
            pltpu.VMEM((KB,), jnp.int32),
            pltpu.VMEM((KB, CF), jnp.float32),
            pltpu.VMEM((KB, CF), jnp.float32),
            pltpu.VMEM((KB, 32), jnp.float32),
            pltpu.VMEM((KB, 32), jnp.float32),
            pltpu.VMEM((KB, CF), jnp.float32),
            pltpu.VMEM((KB, CF), jnp.float32),
            pltpu.SemaphoreType.DMA,
            pltpu.SemaphoreType.DMA,
            pltpu.SemaphoreType.DMA,
            pltpu.SemaphoreType.DMA,
            pltpu.SemaphoreType.DMA,
        ],
    )(fa, fb, ala, alb_, src2, dst, zeros)


def _attn_mats(attn_l, attn_r):
    eye = jnp.eye(HEADS, dtype=jnp.float32)
    Ml = (attn_l[:, :, None] * eye[:, None, :]).reshape(HEADS * FPH, HEADS)
    Mr = (attn_r[:, :, None] * eye[:, None, :]).reshape(HEADS * FPH, HEADS)
    z = jnp.zeros((HEADS * FPH, 128 - HEADS), jnp.float32)
    return jnp.concatenate([Ml, z], 1), jnp.concatenate([Mr, z], 1)


def _rep_mat():
    j = jnp.arange(128)
    head = (j // 32) * 2 + (j % 32) // 16
    return (jnp.arange(NLANE)[:, None] == head[None, :]).astype(jnp.float32)


def _gat_sc_layer(fa, fb, elp, erp, src, src2, dst, zeros, R):
    ex, dp0, dp1 = _sc_softmax_num(elp, erp, src, dst, zeros)
    alpha = _sc_alpha(ex, dp0, dp1, dst)
    ala, alb = _tc_alpha_rep(alpha, R)
    oa, ob = _sc_aggregate(fa.reshape(2 * N, CF), fb.reshape(2 * N, CF),
                           ala.reshape(2 * E, 32), alb.reshape(2 * E, 32),
                           src2, dst, zeros)
    return oa.reshape(2, N, CF), ob.reshape(2, N, CF)


@jax.jit
def kernel(in_feat, edge_index, W1, attn_l1, attn_r1, b1, W2, attn_l2, attn_r2, b2):
    src = edge_index[0]
    dst = edge_index[1]
    src2 = jnp.concatenate([src, src + N])
    ALl1, ALr1 = _attn_mats(attn_l1, attn_r1)
    ALl2, ALr2 = _attn_mats(attn_l2, attn_r2)
    b1r = b1.reshape(1, -1)
    b2r = b2.reshape(1, -1)
    zeros = jnp.zeros((ZROWS, 128), jnp.float32)
    R = _rep_mat()

    fa, fb, elp, erp = _tc_proj(in_feat, W1, ALl1, ALr1)
    oa, ob = _gat_sc_layer(fa, fb, elp, erp, src, src2, dst, zeros, R)

    fa, fb, elp, erp = _tc_elu_proj(oa, ob, b1r, W2, ALl2, ALr2)
    oa, ob = _gat_sc_layer(fa, fb, elp, erp, src, src2, dst, zeros, R)

    return _tc_elu_out(oa, ob, b2r)
